# baseline (speedup 1.0000x reference)
import functools

import jax
import jax.numpy as jnp
from jax.experimental import pallas as pl


BLK_B = 512


def _mm_kernel(counts_ref, emb_ref, out_ref):
    counts = counts_ref[:].astype(jnp.bfloat16)
    emb = emb_ref[:].astype(jnp.bfloat16)
    out_ref[:] = jnp.dot(counts, emb, preferred_element_type=jnp.float32)


@functools.partial(jax.jit, static_argnames=())
def kernel(element_counts, emb):
    B, E = element_counts.shape
    D = emb.shape[1]
    grid = (B // BLK_B,)
    return pl.pallas_call(
        _mm_kernel,
        grid=grid,
        in_specs=[
            pl.BlockSpec((BLK_B, E), lambda i: (i, 0)),
            pl.BlockSpec((E, D), lambda i: (0, 0)),
        ],
        out_specs=pl.BlockSpec((BLK_B, D), lambda i: (i, 0)),
        out_shape=jax.ShapeDtypeStruct((B, D), jnp.float32),
    )(element_counts, emb)


# speedup vs baseline: 1.0685x; 1.0685x over previous
import functools

import jax
import jax.numpy as jnp
from jax.experimental import pallas as pl


BLK_B = 1024


def _mm_kernel(counts_ref, emb_ref, out_ref):
    counts = counts_ref[:].astype(jnp.bfloat16)
    emb = emb_ref[:].astype(jnp.bfloat16)
    out_ref[:] = jnp.dot(counts, emb, preferred_element_type=jnp.float32)


@functools.partial(jax.jit, static_argnames=())
def kernel(element_counts, emb):
    B, E = element_counts.shape
    D = emb.shape[1]
    grid = (B // BLK_B,)
    return pl.pallas_call(
        _mm_kernel,
        grid=grid,
        in_specs=[
            pl.BlockSpec((BLK_B, E), lambda i: (i, 0)),
            pl.BlockSpec((E, D), lambda i: (0, 0)),
        ],
        out_specs=pl.BlockSpec((BLK_B, D), lambda i: (i, 0)),
        out_shape=jax.ShapeDtypeStruct((B, D), jnp.float32),
    )(element_counts, emb)


# speedup vs baseline: 1.0744x; 1.0055x over previous
import functools

import jax
import jax.numpy as jnp
from jax.experimental import pallas as pl


BLK_B = 2048


def _mm_kernel(counts_ref, emb_ref, out_ref):
    counts = counts_ref[:].astype(jnp.bfloat16)
    emb = emb_ref[:].astype(jnp.bfloat16)
    out_ref[:] = jnp.dot(counts, emb, preferred_element_type=jnp.float32)


@functools.partial(jax.jit, static_argnames=())
def kernel(element_counts, emb):
    B, E = element_counts.shape
    D = emb.shape[1]
    grid = (B // BLK_B,)
    return pl.pallas_call(
        _mm_kernel,
        grid=grid,
        in_specs=[
            pl.BlockSpec((BLK_B, E), lambda i: (i, 0)),
            pl.BlockSpec((E, D), lambda i: (0, 0)),
        ],
        out_specs=pl.BlockSpec((BLK_B, D), lambda i: (i, 0)),
        out_shape=jax.ShapeDtypeStruct((B, D), jnp.float32),
    )(element_counts, emb)
